# baseline (speedup 1.0000x reference)
import functools

import jax
import jax.numpy as jnp
from jax import lax
from jax.experimental import pallas as pl
from jax.experimental.pallas import tpu as pltpu
from jax.experimental.pallas import tpu_sc as plsc

VOCAB = 1000000
DIM = 64
BATCH = 4096
SEQ = 200
N_TOK = BATCH * SEQ
LN_EPS = 1e-5

NUM_CORES = 2
NUM_SUBCORES = 16
NW = NUM_CORES * NUM_SUBCORES
PER_W = N_TOK // NW
CHUNK = 1024
K_IDX = CHUNK // 128
N_STEPS = PER_W // CHUNK
UNROLL = 16


def _rsqrt(x):
    i = plsc.bitcast(x, jnp.int32)
    i = jnp.int32(0x5F3759DF) - (i >> 1)
    y = plsc.bitcast(i, jnp.float32)
    xh = x * jnp.float32(0.5)
    for _ in range(3):
        y = y * (jnp.float32(1.5) - xh * y * y)
    return y


def _body(ids_hbm, table_hbm, gamma_hbm, beta_hbm, out_hbm,
          idx_v, rows_v, gvm, bvm, sem):
    wid = lax.axis_index("s") * NUM_CORES + lax.axis_index("c")
    base = wid * PER_W

    pltpu.sync_copy(gamma_hbm, gvm)
    pltpu.sync_copy(beta_hbm, bvm)
    gs = [gvm[pl.ds(16 * cg, 16)] for cg in range(4)]
    bs = [bvm[pl.ds(16 * cg, 16)] for cg in range(4)]

    def step(g, _):
        row0 = base + g * CHUNK
        idx_row0 = pl.multiple_of(row0 // 128, 8)
        pltpu.sync_copy(ids_hbm.at[pl.ds(idx_row0, K_IDX)], idx_v)
        copies = []
        for j in range(K_IDX):
            copies.append(
                pltpu.async_copy(
                    table_hbm.at[idx_v.at[j]],
                    rows_v.at[pl.ds(j * 128, 128)],
                    sem,
                ))
        for cp in copies:
            cp.wait()

        def norm(it, _):
            r0 = it * UNROLL
            for u in range(UNROLL):
                r = r0 + u
                v = [rows_v[r, pl.ds(16 * cg, 16)] for cg in range(4)]
                s = (v[0] + v[1]) + (v[2] + v[3])
                q = (v[0] * v[0] + v[1] * v[1]) + (v[2] * v[2] + v[3] * v[3])
                tot = jnp.sum(s)
                tot2 = jnp.sum(q)
                mean = tot * jnp.float32(1.0 / DIM)
                ex2 = tot2 * jnp.float32(1.0 / DIM)
                var = ex2 - mean * mean
                mean_b = jnp.full((16,), mean, jnp.float32)
                rstd_b = _rsqrt(jnp.full((16,), var + jnp.float32(LN_EPS),
                                         jnp.float32))
                for cg in range(4):
                    o = (v[cg] - mean_b) * rstd_b * gs[cg] + bs[cg]
                    rows_v[r, pl.ds(16 * cg, 16)] = o
            return _

        lax.fori_loop(0, CHUNK // UNROLL, norm, 0)

        pltpu.sync_copy(rows_v, out_hbm.at[pl.ds(row0, CHUNK)])
        return _

    lax.fori_loop(0, N_STEPS, step, 0)


def _run(ids2d, table, gamma, beta):
    mesh = plsc.VectorSubcoreMesh(core_axis_name="c", subcore_axis_name="s")
    k = functools.partial(
        pl.kernel,
        mesh=mesh,
        compiler_params=pltpu.CompilerParams(
            needs_layout_passes=False, use_tc_tiling_on_sc=False
        ),
        out_type=jax.ShapeDtypeStruct((N_TOK, DIM), jnp.float32),
        scratch_types=[
            pltpu.VMEM((K_IDX, 128), jnp.int32),
            pltpu.VMEM((CHUNK, DIM), jnp.float32),
            pltpu.VMEM((DIM,), jnp.float32),
            pltpu.VMEM((DIM,), jnp.float32),
            pltpu.SemaphoreType.DMA,
        ],
    )(_body)
    return k(ids2d, table, gamma, beta)


def kernel(input_ids, table, gamma, beta):
    ids2d = input_ids.reshape(-1).astype(jnp.int32).reshape(N_TOK // 128, 128)
    out = _run(ids2d, table, gamma, beta)
    return out.reshape(BATCH, SEQ, DIM)
